# baseline (speedup 1.0000x reference)
import functools

import jax
import jax.numpy as jnp
from jax import lax
from jax.experimental import pallas as pl
from jax.experimental.pallas import tpu as pltpu
from jax.experimental.pallas import tpu_sc as plsc

N_NODES_PAD = 10240
CHUNK = 128
SUP = 8


def _sc_scatter(t_hbm, negrel_hbm, gidx_hbm,
                part_a, part_c,
                idx_v, buf0, buf1, ones_v,
                a_sh, negrel_sh, c_sh,
                gsem0, gsem1, ssem0, ssem1, csem, isem):
    c = lax.axis_index("c")
    s = lax.axis_index("s")
    wid = c * 16 + s

    n_sup = gidx_hbm.shape[1] // SUP - 1
    rows_per_sub = N_NODES_PAD // 16
    bufs = (buf0, buf1)
    gsems = (gsem0, gsem1)
    ssems = (ssem0, ssem1)

    def _fill_row(i, _):
        for j in range(CHUNK // 16):
            buf1[i, pl.ds(j * 16, 16)] = jnp.zeros((16,), jnp.float32)
        return 0
    lax.fori_loop(0, CHUNK, _fill_row, 0)
    for j in range(CHUNK // 16):
        ones_v[pl.ds(j * 16, 16)] = jnp.ones((16,), jnp.float32)

    base = s * rows_per_sub
    for k in range(rows_per_sub // CHUNK):
        pltpu.sync_copy(buf1, a_sh.at[pl.ds(base + k * CHUNK, CHUNK)])
        pltpu.sync_copy(buf1.at[0], c_sh.at[pl.ds(base + k * CHUNK, CHUNK)])

    @pl.when(s == 0)
    def _stage():
        pltpu.sync_copy(negrel_hbm, negrel_sh)
    plsc.subcore_barrier()

    srcs = (t_hbm, negrel_sh)
    d_i0 = pltpu.async_copy(gidx_hbm.at[wid, pl.ds(0, SUP)], idx_v.at[0],
                            isem)
    del d_i0

    def _outer(o, _):
        p = lax.rem(o, 2)
        ip = idx_v.at[p]
        pltpu.make_async_copy(gidx_hbm.at[wid, pl.ds(o * SUP, SUP)],
                              ip, isem).wait()
        pltpu.async_copy(gidx_hbm.at[wid, pl.ds((o + 1) * SUP, SUP)],
                         idx_v.at[1 - p], isem)
        d_g = [None] * SUP
        d_s = [None] * SUP
        d_c = [None] * SUP
        d_g[0] = pltpu.async_copy(srcs[0].at[ip.at[0, 0]], buf0, gsem0)
        d_g[1] = pltpu.async_copy(srcs[1].at[ip.at[1, 0]], buf1, gsem1)
        for j in range(SUP):
            b = j % 2
            d_c[j] = pltpu.async_copy(ones_v, c_sh.at[ip.at[j, 2]], csem,
                                      add=True)
            d_g[j].wait()
            d_s[j] = pltpu.async_copy(bufs[b], a_sh.at[ip.at[j, 1]],
                                      ssems[b], add=True)
            if j + 2 < SUP:
                d_s[j].wait()
                d_g[j + 2] = pltpu.async_copy(srcs[b].at[ip.at[j + 2, 0]],
                                              bufs[b], gsems[b])
        d_s[SUP - 2].wait()
        d_s[SUP - 1].wait()
        for j in range(SUP):
            d_c[j].wait()
        return 0
    lax.fori_loop(0, n_sup, _outer, 0)
    pltpu.make_async_copy(gidx_hbm.at[wid, pl.ds(n_sup * SUP, SUP)],
                          idx_v.at[lax.rem(n_sup, 2)], isem).wait()
    plsc.subcore_barrier()

    pltpu.sync_copy(a_sh.at[pl.ds(base, rows_per_sub)],
                    part_a.at[c, pl.ds(base, rows_per_sub)])
    pltpu.sync_copy(c_sh.at[pl.ds(base, rows_per_sub)],
                    part_c.at[c, pl.ds(base, rows_per_sub)])


def _tc_finish(pa_ref, pc_ref, wt_ref, out_ref):
    x = pa_ref[0] + pa_ref[1]
    y = jnp.dot(x, wt_ref[...], preferred_element_type=jnp.float32)
    cnt = pc_ref[0] + pc_ref[1]
    out_ref[...] = y / cnt[:, None]


def kernel(node_states, edge_indices, rel_states, W):
    batch, n_nodes, comp_dim = node_states.shape
    out_dim = W.shape[0]
    n_edges = edge_indices.shape[1]
    n_rel = rel_states.shape[0]

    table = node_states[0]
    negrel = jnp.concatenate([-rel_states,
                              jnp.zeros((8, comp_dim), jnp.float32)])

    dst = edge_indices[1]
    src = edge_indices[2]
    rel = edge_indices[3]

    dummy_dst = n_nodes

    hpw = n_edges // 32
    hcpw = -(-hpw // (CHUNK * (SUP // 2))) * (CHUNK * (SUP // 2))
    pad_h = hcpw - hpw
    hsrc = jnp.concatenate([src.reshape(32, -1),
                            jnp.zeros((32, pad_h), jnp.int32)], axis=1)
    hdst = jnp.concatenate([dst.reshape(32, -1),
                            jnp.full((32, pad_h), dummy_dst, jnp.int32)],
                           axis=1)
    rsrc = jnp.roll(rel.reshape(32, -1), hpw // 2, axis=1)
    rdst = jnp.roll(dst.reshape(32, -1), hpw // 2, axis=1)
    rsrc = jnp.concatenate([rsrc,
                            jnp.full((32, pad_h), n_rel, jnp.int32)], axis=1)
    rdst = jnp.concatenate([rdst,
                            jnp.full((32, pad_h), dummy_dst, jnp.int32)],
                           axis=1)
    n_hc = hcpw // CHUNK
    splane = jnp.stack([hsrc.reshape(32, n_hc, CHUNK),
                        rsrc.reshape(32, n_hc, CHUNK)],
                       axis=2).reshape(32, 2 * n_hc, CHUNK)
    dplane = jnp.stack([hdst.reshape(32, n_hc, CHUNK),
                        rdst.reshape(32, n_hc, CHUNK)],
                       axis=2).reshape(32, 2 * n_hc, CHUNK)
    cplane = jnp.concatenate(
        [hdst.reshape(32, n_hc, CHUNK),
         jnp.full((32, n_hc, CHUNK), dummy_dst, jnp.int32)], axis=1)
    gidx = jnp.stack([splane, dplane, cplane], axis=2)
    pad_sup = jnp.full((32, SUP, 3, CHUNK), dummy_dst, jnp.int32)
    pad_sup = pad_sup.at[:, :, 0, :].set(0)
    gidx = jnp.concatenate([gidx, pad_sup], axis=1)

    mesh = plsc.VectorSubcoreMesh(core_axis_name="c", subcore_axis_name="s")
    sc_call = pl.kernel(
        _sc_scatter,
        out_type=[
            jax.ShapeDtypeStruct((2, N_NODES_PAD, comp_dim), jnp.float32),
            jax.ShapeDtypeStruct((2, N_NODES_PAD), jnp.float32),
        ],
        mesh=mesh,
        scratch_types=[
            pltpu.VMEM((2, SUP, 3, CHUNK), jnp.int32),
            pltpu.VMEM((CHUNK, comp_dim), jnp.float32),
            pltpu.VMEM((CHUNK, comp_dim), jnp.float32),
            pltpu.VMEM((CHUNK,), jnp.float32),
            pltpu.VMEM_SHARED((N_NODES_PAD, comp_dim), jnp.float32),
            pltpu.VMEM_SHARED((n_rel + 8, comp_dim), jnp.float32),
            pltpu.VMEM_SHARED((N_NODES_PAD,), jnp.float32),
            pltpu.SemaphoreType.DMA,
            pltpu.SemaphoreType.DMA,
            pltpu.SemaphoreType.DMA,
            pltpu.SemaphoreType.DMA,
            pltpu.SemaphoreType.DMA,
            pltpu.SemaphoreType.DMA,
        ],
    )
    part_a, part_c = sc_call(table, negrel, gidx)

    blk = 1024
    grid = N_NODES_PAD // blk
    out = pl.pallas_call(
        _tc_finish,
        grid=(grid,),
        in_specs=[
            pl.BlockSpec((2, blk, comp_dim), lambda i: (0, i, 0)),
            pl.BlockSpec((2, blk), lambda i: (0, i)),
            pl.BlockSpec((comp_dim, out_dim), lambda i: (0, 0)),
        ],
        out_specs=pl.BlockSpec((blk, out_dim), lambda i: (i, 0)),
        out_shape=jax.ShapeDtypeStruct((N_NODES_PAD, out_dim), jnp.float32),
    )(part_a, part_c, W.T)

    return out[:n_nodes][None]


# speedup vs baseline: 1.1327x; 1.1327x over previous
import functools

import jax
import jax.numpy as jnp
from jax import lax
from jax.experimental import pallas as pl
from jax.experimental.pallas import tpu as pltpu
from jax.experimental.pallas import tpu_sc as plsc

N_NODES_PAD = 10240
CHUNK = 128
SUP = 8


def _sc_scatter(t_hbm, negrel_hbm, gidx_hbm,
                part_a, part_c,
                idx_v, buf0, buf1, ones_v,
                a_sh, negrel_sh, c_sh,
                gsem0, gsem1, ssem0, ssem1, csem, isem):
    c = lax.axis_index("c")
    s = lax.axis_index("s")
    wid = c * 16 + s

    n_sup = gidx_hbm.shape[1] // SUP - 1
    rows_per_sub = N_NODES_PAD // 16
    bufs = (buf0, buf1)
    gsems = (gsem0, gsem1)
    ssems = (ssem0, ssem1)

    def _fill_row(i, _):
        for j in range(CHUNK // 16):
            buf1[i, pl.ds(j * 16, 16)] = jnp.zeros((16,), jnp.float32)
        return 0
    lax.fori_loop(0, CHUNK, _fill_row, 0)
    for j in range(CHUNK // 16):
        ones_v[pl.ds(j * 16, 16)] = jnp.ones((16,), jnp.float32)

    base = s * rows_per_sub
    for k in range(rows_per_sub // CHUNK):
        pltpu.sync_copy(buf1, a_sh.at[pl.ds(base + k * CHUNK, CHUNK)])
        pltpu.sync_copy(buf1.at[0], c_sh.at[pl.ds(base + k * CHUNK, CHUNK)])

    @pl.when(s == 0)
    def _stage():
        pltpu.sync_copy(negrel_hbm, negrel_sh)
    plsc.subcore_barrier()

    srcs = (t_hbm, negrel_sh)
    d_i0 = pltpu.async_copy(gidx_hbm.at[wid, pl.ds(0, SUP)], idx_v.at[0],
                            isem)
    del d_i0

    def _outer(o, _):
        p = lax.rem(o, 2)
        ip = idx_v.at[p]
        pltpu.make_async_copy(gidx_hbm.at[wid, pl.ds(o * SUP, SUP)],
                              ip, isem).wait()
        pltpu.async_copy(gidx_hbm.at[wid, pl.ds((o + 1) * SUP, SUP)],
                         idx_v.at[1 - p], isem)
        d_g = [None] * SUP
        d_s = [None] * SUP
        d_g[0] = pltpu.async_copy(srcs[0].at[ip.at[0, 0]], buf0, gsem0)
        d_g[1] = pltpu.async_copy(srcs[1].at[ip.at[1, 0]], buf1, gsem1)
        for j in range(SUP):
            b = j % 2
            d_g[j].wait()
            d_s[j] = pltpu.async_copy(bufs[b], a_sh.at[ip.at[j, 1]],
                                      ssems[b], add=True)
            if j + 2 < SUP:
                d_s[j].wait()
                d_g[j + 2] = pltpu.async_copy(srcs[b].at[ip.at[j + 2, 0]],
                                              bufs[b], gsems[b])
        d_s[SUP - 2].wait()
        d_s[SUP - 1].wait()
        return 0
    lax.fori_loop(0, n_sup, _outer, 0)
    pltpu.make_async_copy(gidx_hbm.at[wid, pl.ds(n_sup * SUP, SUP)],
                          idx_v.at[lax.rem(n_sup, 2)], isem).wait()

    def _couter(o, _):
        p = lax.rem(o, 2)
        ip = idx_v.at[p]
        pltpu.sync_copy(gidx_hbm.at[wid, pl.ds(o * SUP, SUP)], ip)
        d_c = [pltpu.async_copy(ones_v, c_sh.at[ip.at[j, 2]], csem,
                                add=True)
               for j in range(SUP)]
        for d in d_c:
            d.wait()
        return 0
    lax.fori_loop(0, n_sup // 2, _couter, 0)
    plsc.subcore_barrier()

    pltpu.sync_copy(a_sh.at[pl.ds(base, rows_per_sub)],
                    part_a.at[c, pl.ds(base, rows_per_sub)])
    pltpu.sync_copy(c_sh.at[pl.ds(base, rows_per_sub)],
                    part_c.at[c, pl.ds(base, rows_per_sub)])


def _tc_finish(pa_ref, pc_ref, wt_ref, out_ref):
    x = pa_ref[0] + pa_ref[1]
    y = jnp.dot(x, wt_ref[...], preferred_element_type=jnp.float32)
    cnt = pc_ref[0] + pc_ref[1]
    out_ref[...] = y / cnt[:, None]


def kernel(node_states, edge_indices, rel_states, W):
    batch, n_nodes, comp_dim = node_states.shape
    out_dim = W.shape[0]
    n_edges = edge_indices.shape[1]
    n_rel = rel_states.shape[0]

    table = node_states[0]
    negrel = jnp.concatenate([-rel_states,
                              jnp.zeros((8, comp_dim), jnp.float32)])

    dst = edge_indices[1]
    src = edge_indices[2]
    rel = edge_indices[3]

    dummy_dst = n_nodes

    hpw = n_edges // 32
    hcpw = -(-hpw // (CHUNK * (SUP // 2))) * (CHUNK * (SUP // 2))
    pad_h = hcpw - hpw
    hsrc = jnp.concatenate([src.reshape(32, -1),
                            jnp.zeros((32, pad_h), jnp.int32)], axis=1)
    hdst = jnp.concatenate([dst.reshape(32, -1),
                            jnp.full((32, pad_h), dummy_dst, jnp.int32)],
                           axis=1)
    rsrc = jnp.roll(rel.reshape(32, -1), hpw // 2, axis=1)
    rdst = jnp.roll(dst.reshape(32, -1), hpw // 2, axis=1)
    rsrc = jnp.concatenate([rsrc,
                            jnp.full((32, pad_h), n_rel, jnp.int32)], axis=1)
    rdst = jnp.concatenate([rdst,
                            jnp.full((32, pad_h), dummy_dst, jnp.int32)],
                           axis=1)
    n_hc = hcpw // CHUNK
    splane = jnp.stack([hsrc.reshape(32, n_hc, CHUNK),
                        rsrc.reshape(32, n_hc, CHUNK)],
                       axis=2).reshape(32, 2 * n_hc, CHUNK)
    dplane = jnp.stack([hdst.reshape(32, n_hc, CHUNK),
                        rdst.reshape(32, n_hc, CHUNK)],
                       axis=2).reshape(32, 2 * n_hc, CHUNK)
    cplane = jnp.concatenate(
        [hdst.reshape(32, n_hc, CHUNK),
         jnp.full((32, n_hc, CHUNK), dummy_dst, jnp.int32)], axis=1)
    gidx = jnp.stack([splane, dplane, cplane], axis=2)
    pad_sup = jnp.full((32, SUP, 3, CHUNK), dummy_dst, jnp.int32)
    pad_sup = pad_sup.at[:, :, 0, :].set(0)
    gidx = jnp.concatenate([gidx, pad_sup], axis=1)

    mesh = plsc.VectorSubcoreMesh(core_axis_name="c", subcore_axis_name="s")
    sc_call = pl.kernel(
        _sc_scatter,
        out_type=[
            jax.ShapeDtypeStruct((2, N_NODES_PAD, comp_dim), jnp.float32),
            jax.ShapeDtypeStruct((2, N_NODES_PAD), jnp.float32),
        ],
        mesh=mesh,
        scratch_types=[
            pltpu.VMEM((2, SUP, 3, CHUNK), jnp.int32),
            pltpu.VMEM((CHUNK, comp_dim), jnp.float32),
            pltpu.VMEM((CHUNK, comp_dim), jnp.float32),
            pltpu.VMEM((CHUNK,), jnp.float32),
            pltpu.VMEM_SHARED((N_NODES_PAD, comp_dim), jnp.float32),
            pltpu.VMEM_SHARED((n_rel + 8, comp_dim), jnp.float32),
            pltpu.VMEM_SHARED((N_NODES_PAD,), jnp.float32),
            pltpu.SemaphoreType.DMA,
            pltpu.SemaphoreType.DMA,
            pltpu.SemaphoreType.DMA,
            pltpu.SemaphoreType.DMA,
            pltpu.SemaphoreType.DMA,
            pltpu.SemaphoreType.DMA,
        ],
    )
    part_a, part_c = sc_call(table, negrel, gidx)

    blk = 1024
    grid = N_NODES_PAD // blk
    out = pl.pallas_call(
        _tc_finish,
        grid=(grid,),
        in_specs=[
            pl.BlockSpec((2, blk, comp_dim), lambda i: (0, i, 0)),
            pl.BlockSpec((2, blk), lambda i: (0, i)),
            pl.BlockSpec((comp_dim, out_dim), lambda i: (0, 0)),
        ],
        out_specs=pl.BlockSpec((blk, out_dim), lambda i: (i, 0)),
        out_shape=jax.ShapeDtypeStruct((N_NODES_PAD, out_dim), jnp.float32),
    )(part_a, part_c, W.T)

    return out[:n_nodes][None]


# speedup vs baseline: 1.1399x; 1.0063x over previous
import jax
import jax.numpy as jnp
from jax import lax
from jax.experimental import pallas as pl
from jax.experimental.pallas import tpu as pltpu
from jax.experimental.pallas import tpu_sc as plsc

N_NODES_PAD = 10240
CHUNK = 128
SUP = 8


def _sc_scatter(t_hbm, negrel_hbm, gidx_hbm,
                part_a, part_c,
                idx_v, buf0, buf1, ones_v,
                a_sh, negrel_sh, c_sh,
                gsem0, gsem1, ssem0, ssem1, csem, isem):
    c = lax.axis_index("c")
    s = lax.axis_index("s")
    wid = c * 16 + s

    n_sup = gidx_hbm.shape[1] // SUP - 1
    rows_per_sub = N_NODES_PAD // 16
    bufs = (buf0, buf1)
    gsems = (gsem0, gsem1)
    ssems = (ssem0, ssem1)

    def _fill_row(i, _):
        for j in range(CHUNK // 16):
            buf1[i, pl.ds(j * 16, 16)] = jnp.zeros((16,), jnp.float32)
        return 0
    lax.fori_loop(0, CHUNK, _fill_row, 0)
    for j in range(CHUNK // 16):
        ones_v[pl.ds(j * 16, 16)] = jnp.ones((16,), jnp.float32)

    base = s * rows_per_sub
    for k in range(rows_per_sub // CHUNK):
        pltpu.sync_copy(buf1, a_sh.at[pl.ds(base + k * CHUNK, CHUNK)])
        pltpu.sync_copy(buf1.at[0], c_sh.at[pl.ds(base + k * CHUNK, CHUNK)])

    @pl.when(s == 0)
    def _stage():
        pltpu.sync_copy(negrel_hbm, negrel_sh)
    plsc.subcore_barrier()

    srcs = (t_hbm, negrel_sh)
    d_i0 = pltpu.async_copy(gidx_hbm.at[wid, pl.ds(0, SUP)], idx_v.at[0],
                            isem)
    del d_i0

    def _outer(o, _):
        p = lax.rem(o, 2)
        ip = idx_v.at[p]
        pltpu.make_async_copy(gidx_hbm.at[wid, pl.ds(o * SUP, SUP)],
                              ip, isem).wait()
        pltpu.async_copy(gidx_hbm.at[wid, pl.ds((o + 1) * SUP, SUP)],
                         idx_v.at[1 - p], isem)
        d_g = [None] * SUP
        d_s = [None] * SUP
        d_g[0] = pltpu.async_copy(srcs[0].at[ip.at[0, 0]], buf0, gsem0)
        d_g[1] = pltpu.async_copy(srcs[1].at[ip.at[1, 0]], buf1, gsem1)
        for j in range(SUP):
            b = j % 2
            d_g[j].wait()
            d_s[j] = pltpu.async_copy(bufs[b], a_sh.at[ip.at[j, 1]],
                                      ssems[b], add=True)
            if j + 2 < SUP:
                d_s[j].wait()
                d_g[j + 2] = pltpu.async_copy(srcs[b].at[ip.at[j + 2, 0]],
                                              bufs[b], gsems[b])
        d_s[SUP - 2].wait()
        d_s[SUP - 1].wait()
        return 0
    lax.fori_loop(0, n_sup, _outer, 0)
    pltpu.make_async_copy(gidx_hbm.at[wid, pl.ds(n_sup * SUP, SUP)],
                          idx_v.at[lax.rem(n_sup, 2)], isem).wait()

    def _couter(o, _):
        p = lax.rem(o, 2)
        ip = idx_v.at[p]
        pltpu.sync_copy(gidx_hbm.at[wid, pl.ds(o * SUP, SUP)], ip)
        d_c = [pltpu.async_copy(ones_v, c_sh.at[ip.at[j, 2]], csem,
                                add=True)
               for j in range(SUP)]
        for d in d_c:
            d.wait()
        return 0
    lax.fori_loop(0, n_sup // 2, _couter, 0)
    plsc.subcore_barrier()

    pltpu.sync_copy(a_sh.at[pl.ds(base, rows_per_sub)],
                    part_a.at[c, pl.ds(base, rows_per_sub)])
    pltpu.sync_copy(c_sh.at[pl.ds(base, rows_per_sub)],
                    part_c.at[c, pl.ds(base, rows_per_sub)])


def _tc_finish(pa_ref, pc_ref, wt_ref, out_ref):
    x = pa_ref[0] + pa_ref[1]
    y = jnp.dot(x, wt_ref[...], preferred_element_type=jnp.float32)
    cnt = pc_ref[0] + pc_ref[1]
    out_ref[...] = y / cnt[:, None]


def kernel(node_states, edge_indices, rel_states, W):
    batch, n_nodes, comp_dim = node_states.shape
    out_dim = W.shape[0]
    n_edges = edge_indices.shape[1]
    n_rel = rel_states.shape[0]

    table = node_states[0]
    negrel = jnp.concatenate([-rel_states,
                              jnp.zeros((8, comp_dim), jnp.float32)])

    dst = edge_indices[1]
    src = edge_indices[2]
    rel = edge_indices[3]

    dummy_dst = n_nodes

    hpw = n_edges // 32
    hcpw = -(-hpw // (CHUNK * (SUP // 2))) * (CHUNK * (SUP // 2))
    pad_h = hcpw - hpw
    hsrc = jnp.concatenate([src.reshape(32, -1),
                            jnp.zeros((32, pad_h), jnp.int32)], axis=1)
    hdst = jnp.concatenate([dst.reshape(32, -1),
                            jnp.full((32, pad_h), dummy_dst, jnp.int32)],
                           axis=1)
    rsrc = jnp.roll(rel.reshape(32, -1), hpw // 2, axis=1)
    rdst = jnp.roll(dst.reshape(32, -1), hpw // 2, axis=1)
    rsrc = jnp.concatenate([rsrc,
                            jnp.full((32, pad_h), n_rel, jnp.int32)], axis=1)
    rdst = jnp.concatenate([rdst,
                            jnp.full((32, pad_h), dummy_dst, jnp.int32)],
                           axis=1)
    n_hc = hcpw // CHUNK
    splane = jnp.stack([hsrc.reshape(32, n_hc, CHUNK),
                        rsrc.reshape(32, n_hc, CHUNK)],
                       axis=2).reshape(32, 2 * n_hc, CHUNK)
    dplane = jnp.stack([hdst.reshape(32, n_hc, CHUNK),
                        rdst.reshape(32, n_hc, CHUNK)],
                       axis=2).reshape(32, 2 * n_hc, CHUNK)
    cplane = jnp.concatenate(
        [hdst.reshape(32, n_hc, CHUNK),
         jnp.full((32, n_hc, CHUNK), dummy_dst, jnp.int32)], axis=1)
    gidx = jnp.stack([splane, dplane, cplane], axis=2)
    pad_sup = jnp.full((32, SUP, 3, CHUNK), dummy_dst, jnp.int32)
    pad_sup = pad_sup.at[:, :, 0, :].set(0)
    gidx = jnp.concatenate([gidx, pad_sup], axis=1)

    mesh = plsc.VectorSubcoreMesh(core_axis_name="c", subcore_axis_name="s")
    sc_call = pl.kernel(
        _sc_scatter,
        out_type=[
            jax.ShapeDtypeStruct((2, N_NODES_PAD, comp_dim), jnp.float32),
            jax.ShapeDtypeStruct((2, N_NODES_PAD), jnp.float32),
        ],
        mesh=mesh,
        scratch_types=[
            pltpu.VMEM((2, SUP, 3, CHUNK), jnp.int32),
            pltpu.VMEM((CHUNK, comp_dim), jnp.float32),
            pltpu.VMEM((CHUNK, comp_dim), jnp.float32),
            pltpu.VMEM((CHUNK,), jnp.float32),
            pltpu.VMEM_SHARED((N_NODES_PAD, comp_dim), jnp.float32),
            pltpu.VMEM_SHARED((n_rel + 8, comp_dim), jnp.float32),
            pltpu.VMEM_SHARED((N_NODES_PAD,), jnp.float32),
            pltpu.SemaphoreType.DMA,
            pltpu.SemaphoreType.DMA,
            pltpu.SemaphoreType.DMA,
            pltpu.SemaphoreType.DMA,
            pltpu.SemaphoreType.DMA,
            pltpu.SemaphoreType.DMA,
        ],
    )
    part_a, part_c = sc_call(table, negrel, gidx)

    blk = 1024
    grid = N_NODES_PAD // blk
    out = pl.pallas_call(
        _tc_finish,
        grid=(grid,),
        in_specs=[
            pl.BlockSpec((2, blk, comp_dim), lambda i: (0, i, 0)),
            pl.BlockSpec((2, blk), lambda i: (0, i)),
            pl.BlockSpec((comp_dim, out_dim), lambda i: (0, 0)),
        ],
        out_specs=pl.BlockSpec((blk, out_dim), lambda i: (i, 0)),
        out_shape=jax.ShapeDtypeStruct((N_NODES_PAD, out_dim), jnp.float32),
    )(part_a, part_c, W.T)

    return out[:n_nodes][None]
